# baseline (speedup 1.0000x reference)
import functools

import jax
import jax.numpy as jnp
from jax.experimental import pallas as pl
from jax.experimental.pallas import tpu as pltpu

_LANES = 128
_SUBLANES = 8
_TILE_ROWS = 16384
_NUM_CORES = 2


def _ssq_kernel(yhat_ref, y_ref, out_ref, acc_ref, *,
                rows, tile_rows, tiles_per_core, masked):
    c = pl.program_id(0)
    i = pl.program_id(1)

    @pl.when(i == 0)
    def _():
        acc_ref[...] = jnp.zeros_like(acc_ref)

    d = yhat_ref[...].astype(jnp.float32) - y_ref[...].astype(jnp.float32)
    sq = d * d

    if masked:
        start = (c * tiles_per_core + i) * tile_rows
        rid = jax.lax.broadcasted_iota(jnp.int32, sq.shape, 0)
        sq = jnp.where(start + rid < rows, sq, 0.0)

    acc_ref[...] += jnp.sum(
        sq.reshape(tile_rows // _SUBLANES, _SUBLANES, _LANES), axis=0)

    @pl.when(i == tiles_per_core - 1)
    def _():
        s = jnp.sum(acc_ref[...])
        out_ref[...] = jnp.broadcast_to(s, out_ref.shape)


def kernel(yhat, y):
    assert yhat.shape == y.shape, "yhat and y must have the same shape"
    total_n = yhat.size
    if total_n == 0:
        return jnp.float32(jnp.nan)

    yh = jnp.ravel(yhat)
    yy = jnp.ravel(y)

    pad = (-total_n) % (_SUBLANES * _LANES)
    if pad:
        yh = jnp.pad(yh, (0, pad))
        yy = jnp.pad(yy, (0, pad))
    rows = (total_n + pad) // _LANES
    yh2 = yh.reshape(rows, _LANES)
    yy2 = yy.reshape(rows, _LANES)

    tile_rows = min(_TILE_ROWS, rows)
    n_tiles = pl.cdiv(rows, tile_rows)
    num_cores = min(_NUM_CORES, n_tiles)
    tiles_per_core = pl.cdiv(n_tiles, num_cores)
    masked = (rows % tile_rows != 0) or (num_cores * tiles_per_core != n_tiles)

    if masked:
        def in_map(c, i):
            return (jnp.minimum(c * tiles_per_core + i, n_tiles - 1), 0)
    else:
        def in_map(c, i):
            return (c * tiles_per_core + i, 0)

    body = functools.partial(
        _ssq_kernel,
        rows=rows,
        tile_rows=tile_rows,
        tiles_per_core=tiles_per_core,
        masked=masked,
    )

    partials = pl.pallas_call(
        body,
        out_shape=jax.ShapeDtypeStruct((num_cores * _SUBLANES, _LANES),
                                       jnp.float32),
        grid=(num_cores, tiles_per_core),
        in_specs=[
            pl.BlockSpec((tile_rows, _LANES), in_map),
            pl.BlockSpec((tile_rows, _LANES), in_map),
        ],
        out_specs=pl.BlockSpec((_SUBLANES, _LANES), lambda c, i: (c, 0)),
        scratch_shapes=[pltpu.VMEM((_SUBLANES, _LANES), jnp.float32)],
        compiler_params=pltpu.CompilerParams(
            dimension_semantics=("parallel", "arbitrary"),
        ),
    )(yh2, yy2)

    total_sq = jnp.sum(partials[::_SUBLANES, 0])
    return jnp.sqrt(total_sq / jnp.float32(total_n))


# speedup vs baseline: 4.0335x; 4.0335x over previous
import functools

import jax
import jax.numpy as jnp
from jax.experimental import pallas as pl
from jax.experimental.pallas import tpu as pltpu

_LANES = 128
_SUBLANES = 8
_BLOCK_BYTES = 8 * 1024 * 1024


def _ssq_kernel(yhat_ref, y_ref, out_ref, acc_ref, *,
                rows, block_rows, inv_n, masked):
    i = pl.program_id(0)

    @pl.when(i == 0)
    def _():
        acc_ref[...] = jnp.zeros_like(acc_ref)

    d = yhat_ref[...].astype(jnp.float32) - y_ref[...].astype(jnp.float32)
    sq = d * d

    if masked:
        rid = jax.lax.broadcasted_iota(jnp.int32, sq.shape, 0)
        sq = jnp.where(i * block_rows + rid < rows, sq, 0.0)

    acc_ref[...] += jnp.sum(
        sq.reshape(block_rows // _SUBLANES, _SUBLANES, sq.shape[-1]), axis=0)

    @pl.when(i == pl.num_programs(0) - 1)
    def _():
        s = jnp.sum(acc_ref[...])
        out_ref[...] = jnp.broadcast_to(jnp.sqrt(s * inv_n), out_ref.shape)


def _streamed_rmse(yh2, yy2, total_n):
    rows, t = yh2.shape

    block_rows = max(_SUBLANES,
                     min(rows, _BLOCK_BYTES // (t * 4)) // _SUBLANES * _SUBLANES)
    n_blocks = pl.cdiv(rows, block_rows)
    masked = rows % block_rows != 0

    if masked:
        def in_map(i):
            return (jnp.minimum(i, n_blocks - 1), 0)
    else:
        def in_map(i):
            return (i, 0)

    body = functools.partial(
        _ssq_kernel,
        rows=rows,
        block_rows=block_rows,
        inv_n=float(1.0 / total_n),
        masked=masked,
    )

    out = pl.pallas_call(
        body,
        out_shape=jax.ShapeDtypeStruct((_SUBLANES, _LANES), jnp.float32),
        grid=(n_blocks,),
        in_specs=[
            pl.BlockSpec((block_rows, t), in_map),
            pl.BlockSpec((block_rows, t), in_map),
        ],
        out_specs=pl.BlockSpec((_SUBLANES, _LANES), lambda i: (0, 0)),
        scratch_shapes=[pltpu.VMEM((_SUBLANES, t), jnp.float32)],
        compiler_params=pltpu.CompilerParams(
            dimension_semantics=("arbitrary",),
        ),
    )(yh2, yy2)

    return out[0, 0]


def kernel(yhat, y):
    assert yhat.shape == y.shape, "yhat and y must have the same shape"
    total_n = yhat.size
    if total_n == 0:
        return jnp.float32(jnp.nan)

    shape = yhat.shape
    if (len(shape) >= 2 and shape[-1] % _LANES == 0
            and shape[-2] % _SUBLANES == 0):
        t = shape[-1]
        yh2 = yhat.reshape(total_n // t, t)
        yy2 = y.reshape(total_n // t, t)
        return _streamed_rmse(yh2, yy2, total_n)

    yh = jnp.ravel(yhat)
    yy = jnp.ravel(y)
    pad = (-total_n) % (_SUBLANES * _LANES)
    if pad:
        yh = jnp.pad(yh, (0, pad))
        yy = jnp.pad(yy, (0, pad))
    rows = (total_n + pad) // _LANES
    return _streamed_rmse(yh.reshape(rows, _LANES), yy.reshape(rows, _LANES),
                          total_n)


# speedup vs baseline: 4.1110x; 1.0192x over previous
import functools

import jax
import jax.numpy as jnp
from jax.experimental import pallas as pl
from jax.experimental.pallas import tpu as pltpu

_LANES = 128
_SUBLANES = 8
_BLOCK_BYTES = 4 * 1024 * 1024


def _ssq_kernel(yhat_ref, y_ref, out_ref, acc_ref, *,
                rows, block_rows, inv_n, masked):
    i = pl.program_id(0)

    @pl.when(i == 0)
    def _():
        acc_ref[...] = jnp.zeros_like(acc_ref)

    d = yhat_ref[...].astype(jnp.float32) - y_ref[...].astype(jnp.float32)
    sq = d * d

    if masked:
        rid = jax.lax.broadcasted_iota(jnp.int32, sq.shape, 0)
        sq = jnp.where(i * block_rows + rid < rows, sq, 0.0)

    acc_ref[...] += jnp.sum(
        sq.reshape(block_rows // _SUBLANES, _SUBLANES, sq.shape[-1]), axis=0)

    @pl.when(i == pl.num_programs(0) - 1)
    def _():
        s = jnp.sum(acc_ref[...])
        out_ref[...] = jnp.broadcast_to(jnp.sqrt(s * inv_n), out_ref.shape)


def _streamed_rmse(yh2, yy2, total_n):
    rows, t = yh2.shape

    block_rows = max(_SUBLANES,
                     min(rows, _BLOCK_BYTES // (t * 4)) // _SUBLANES * _SUBLANES)
    n_blocks = pl.cdiv(rows, block_rows)
    masked = rows % block_rows != 0

    if masked:
        def in_map(i):
            return (jnp.minimum(i, n_blocks - 1), 0)
    else:
        def in_map(i):
            return (i, 0)

    body = functools.partial(
        _ssq_kernel,
        rows=rows,
        block_rows=block_rows,
        inv_n=float(1.0 / total_n),
        masked=masked,
    )

    out = pl.pallas_call(
        body,
        out_shape=jax.ShapeDtypeStruct((_SUBLANES, _LANES), jnp.float32),
        grid=(n_blocks,),
        in_specs=[
            pl.BlockSpec((block_rows, t), in_map),
            pl.BlockSpec((block_rows, t), in_map),
        ],
        out_specs=pl.BlockSpec((_SUBLANES, _LANES), lambda i: (0, 0)),
        scratch_shapes=[pltpu.VMEM((_SUBLANES, t), jnp.float32)],
        compiler_params=pltpu.CompilerParams(
            dimension_semantics=("arbitrary",),
        ),
    )(yh2, yy2)

    return out[0, 0]


def kernel(yhat, y):
    assert yhat.shape == y.shape, "yhat and y must have the same shape"
    total_n = yhat.size
    if total_n == 0:
        return jnp.float32(jnp.nan)

    shape = yhat.shape
    if (len(shape) >= 2 and shape[-1] % _LANES == 0
            and shape[-2] % _SUBLANES == 0):
        t = shape[-1]
        yh2 = yhat.reshape(total_n // t, t)
        yy2 = y.reshape(total_n // t, t)
        return _streamed_rmse(yh2, yy2, total_n)

    yh = jnp.ravel(yhat)
    yy = jnp.ravel(y)
    pad = (-total_n) % (_SUBLANES * _LANES)
    if pad:
        yh = jnp.pad(yh, (0, pad))
        yy = jnp.pad(yy, (0, pad))
    rows = (total_n + pad) // _LANES
    return _streamed_rmse(yh.reshape(rows, _LANES), yy.reshape(rows, _LANES),
                          total_n)
